# baseline (speedup 1.0000x reference)
import functools

import jax
import jax.numpy as jnp
from jax import lax
from jax.experimental import pallas as pl
from jax.experimental.pallas import tpu as pltpu
from jax.experimental.pallas import tpu_sc as plsc

CHUNK = 128
NUM_CORES = 2
NUM_SUBCORES = 16
NW = NUM_CORES * NUM_SUBCORES
N_PAD = 10240
N_PAD_DEG = 16384


def _sc_degree(dst2d, zeros_np):
    nch = dst2d.shape[0]
    cpw = nch // NW
    rps = N_PAD_DEG // NUM_SUBCORES
    mesh = plsc.VectorSubcoreMesh(core_axis_name="c", subcore_axis_name="s")

    @functools.partial(
        pl.kernel,
        mesh=mesh,
        out_type=jax.ShapeDtypeStruct((NUM_CORES * N_PAD_DEG,), jnp.float32),
        compiler_params=pltpu.CompilerParams(use_tc_tiling_on_sc=False),
        scratch_types=[
            pltpu.VMEM((cpw, CHUNK), jnp.int32),
            pltpu.VMEM((CHUNK,), jnp.float32),
            pltpu.VMEM_SHARED((N_PAD_DEG,), jnp.float32),
            pltpu.SemaphoreType.DMA,
        ],
    )
    def deg_kernel(dst_hbm, z_hbm, out_hbm, dst_v, ones_v, acc, sem):
        c = lax.axis_index("c")
        s = lax.axis_index("s")
        wid = s * NUM_CORES + c
        pltpu.sync_copy(z_hbm.at[pl.ds(s * rps, rps)], acc.at[pl.ds(s * rps, rps)])
        pltpu.sync_copy(dst_hbm.at[pl.ds(wid * cpw, cpw)], dst_v)
        for i in range(CHUNK // 16):
            ones_v[pl.ds(i * 16, 16)] = jnp.full((16,), 1.0, jnp.float32)
        plsc.subcore_barrier()

        def fire(j, carry):
            pltpu.make_async_copy(ones_v, acc.at[dst_v.at[j]], sem).start(add=True)
            return carry

        lax.fori_loop(0, cpw, fire, 0)

        def drain(j, carry):
            pltpu.make_async_copy(ones_v, acc.at[pl.ds(0, CHUNK)], sem).wait()
            return carry

        lax.fori_loop(0, cpw, drain, 0)
        plsc.subcore_barrier()
        pltpu.sync_copy(acc.at[pl.ds(s * rps, rps)],
                        out_hbm.at[pl.ds(c * N_PAD_DEG + s * rps, rps)])

    return deg_kernel(dst2d, zeros_np)


def _sc_aggregate(hs, src2d, dst2d, zeros_nf):
    n, f = hs.shape
    nch = src2d.shape[0]
    cpw = nch // NW
    epw = cpw * CHUNK
    rps = N_PAD // NUM_SUBCORES
    mesh = plsc.VectorSubcoreMesh(core_axis_name="c", subcore_axis_name="s")

    @functools.partial(
        pl.kernel,
        mesh=mesh,
        out_type=jax.ShapeDtypeStruct((NUM_CORES, N_PAD, f), jnp.float32),
        compiler_params=pltpu.CompilerParams(use_tc_tiling_on_sc=False),
        scratch_types=[
            pltpu.VMEM((cpw, CHUNK), jnp.int32),
            pltpu.VMEM((cpw, CHUNK), jnp.int32),
            pltpu.VMEM((cpw * CHUNK, f), jnp.float32),
            pltpu.VMEM_SHARED((N_PAD, f), jnp.float32),
            pltpu.SemaphoreType.DMA,
            pltpu.SemaphoreType.DMA,
        ],
    )
    def agg_kernel(hs_hbm, src_hbm, dst_hbm, z_hbm, out_hbm,
                   src_v, dst_v, rows_v, acc, gsem, ssem):
        c = lax.axis_index("c")
        s = lax.axis_index("s")
        wid = s * NUM_CORES + c
        pltpu.sync_copy(z_hbm.at[pl.ds(s * rps, rps)], acc.at[pl.ds(s * rps, rps)])
        pltpu.sync_copy(src_hbm.at[pl.ds(wid * cpw, cpw)], src_v)
        pltpu.sync_copy(dst_hbm.at[pl.ds(wid * cpw, cpw)], dst_v)
        plsc.subcore_barrier()

        def fire_gather(j, carry):
            pltpu.make_async_copy(
                hs_hbm.at[src_v.at[j]],
                rows_v.at[pl.ds(j * CHUNK, CHUNK), :], gsem).start()
            return carry

        lax.fori_loop(0, cpw, fire_gather, 0)

        def drain_gather(j, carry):
            pltpu.make_async_copy(hs_hbm.at[pl.ds(0, CHUNK), :],
                                  rows_v.at[pl.ds(0, CHUNK), :], gsem).wait()
            return carry

        lax.fori_loop(0, cpw, drain_gather, 0)

        def fire_scatter(j, carry):
            pltpu.make_async_copy(
                rows_v.at[pl.ds(j * CHUNK, CHUNK), :],
                acc.at[dst_v.at[j]], ssem).start(add=True)
            return carry

        lax.fori_loop(0, cpw, fire_scatter, 0)

        def drain_scatter(j, carry):
            pltpu.make_async_copy(rows_v.at[pl.ds(0, CHUNK), :],
                                  acc.at[pl.ds(0, CHUNK), :], ssem).wait()
            return carry

        lax.fori_loop(0, cpw, drain_scatter, 0)
        plsc.subcore_barrier()
        pltpu.sync_copy(acc.at[pl.ds(s * rps, rps), :],
                        out_hbm.at[c, pl.ds(s * rps, rps), :])

    return agg_kernel(hs, src2d, dst2d, zeros_nf)


def _tc_layer1(x, w1, d0, d1, block_n=2000):
    n, f_in = x.shape
    hid = w1.shape[1]
    grid = n // block_n

    def body(x_ref, w_ref, d0_ref, d1_ref, hs_ref, dinv_ref):
        deg = d0_ref[...] + d1_ref[...] + 1.0
        dinv = lax.rsqrt(deg)
        hw = jnp.dot(x_ref[...], w_ref[...], preferred_element_type=jnp.float32)
        hs_ref[...] = hw * dinv
        dinv_ref[...] = dinv

    return pl.pallas_call(
        body,
        grid=(grid,),
        in_specs=[
            pl.BlockSpec((block_n, f_in), lambda i: (i, 0)),
            pl.BlockSpec((f_in, hid), lambda i: (0, 0)),
            pl.BlockSpec((block_n, 1), lambda i: (i, 0)),
            pl.BlockSpec((block_n, 1), lambda i: (i, 0)),
        ],
        out_specs=[
            pl.BlockSpec((block_n, hid), lambda i: (i, 0)),
            pl.BlockSpec((block_n, 1), lambda i: (i, 0)),
        ],
        out_shape=[
            jax.ShapeDtypeStruct((n, hid), jnp.float32),
            jax.ShapeDtypeStruct((n, 1), jnp.float32),
        ],
    )(x, w1, d0, d1)


def _tc_layer2(a0, a1, hs1, dinv, b1, block_n=2000):
    n, hid = hs1.shape
    grid = n // block_n

    def body(a0_ref, a1_ref, hs_ref, dinv_ref, b1_ref, out_ref):
        dinv = dinv_ref[...]
        t = dinv * (a0_ref[...] + a1_ref[...] + hs_ref[...]) + b1_ref[...]
        out_ref[...] = jnp.maximum(t, 0.0) * dinv

    return pl.pallas_call(
        body,
        grid=(grid,),
        in_specs=[
            pl.BlockSpec((block_n, hid), lambda i: (i, 0)),
            pl.BlockSpec((block_n, hid), lambda i: (i, 0)),
            pl.BlockSpec((block_n, hid), lambda i: (i, 0)),
            pl.BlockSpec((block_n, 1), lambda i: (i, 0)),
            pl.BlockSpec((1, hid), lambda i: (0, 0)),
        ],
        out_specs=pl.BlockSpec((block_n, hid), lambda i: (i, 0)),
        out_shape=jax.ShapeDtypeStruct((n, hid), jnp.float32),
    )(a0, a1, hs1, dinv, b1)


def _tc_final(q0, q1, hs2, dinv, w2, b2, block_n=2000):
    n, hid = hs2.shape
    c_out = w2.shape[1]
    grid = n // block_n

    def body(q0_ref, q1_ref, hs_ref, dinv_ref, w2_ref, b2_ref, out_ref):
        t = dinv_ref[...] * (q0_ref[...] + q1_ref[...] + hs_ref[...])
        logits = jnp.dot(t, w2_ref[...],
                         preferred_element_type=jnp.float32) + b2_ref[...]
        m = jnp.max(logits, axis=1, keepdims=True)
        lse = jnp.log(jnp.sum(jnp.exp(logits - m), axis=1, keepdims=True)) + m
        out_ref[...] = logits - lse

    return pl.pallas_call(
        body,
        grid=(grid,),
        in_specs=[
            pl.BlockSpec((block_n, hid), lambda i: (i, 0)),
            pl.BlockSpec((block_n, hid), lambda i: (i, 0)),
            pl.BlockSpec((block_n, hid), lambda i: (i, 0)),
            pl.BlockSpec((block_n, 1), lambda i: (i, 0)),
            pl.BlockSpec((hid, c_out), lambda i: (0, 0)),
            pl.BlockSpec((1, c_out), lambda i: (0, 0)),
        ],
        out_specs=pl.BlockSpec((block_n, c_out), lambda i: (i, 0)),
        out_shape=jax.ShapeDtypeStruct((n, c_out), jnp.float32),
    )(q0, q1, hs2, dinv, w2, b2)


def kernel(x, edge_index, W1, b1, W2, b2):
    n, f_in = x.shape
    e = edge_index.shape[1]
    hid = W1.shape[1]
    c_out = W2.shape[1]

    e_pad = ((e + CHUNK * NW - 1) // (CHUNK * NW)) * (CHUNK * NW)
    pad = e_pad - e
    src_full = jnp.concatenate([edge_index[0], jnp.zeros((pad,), jnp.int32)])
    dst_full = jnp.concatenate(
        [edge_index[1], jnp.full((pad,), n, jnp.int32)])
    src2d = src_full.reshape(e_pad // CHUNK, CHUNK)
    dst2d = dst_full.reshape(e_pad // CHUNK, CHUNK)

    zeros_np = jnp.zeros((N_PAD_DEG,), jnp.float32)
    zeros_nh = jnp.zeros((N_PAD, hid), jnp.float32)
    b1r = b1.reshape(1, hid)
    b2r = b2.reshape(1, c_out)

    degp = _sc_degree(dst2d, zeros_np)
    d0 = degp[:n].reshape(n, 1)
    d1 = degp[N_PAD_DEG:N_PAD_DEG + n].reshape(n, 1)

    hs1, dinv = _tc_layer1(x, W1, d0, d1)
    p = _sc_aggregate(hs1, src2d, dst2d, zeros_nh)
    hs2 = _tc_layer2(p[0, :n], p[1, :n], hs1, dinv, b1r)
    q = _sc_aggregate(hs2, src2d, dst2d, zeros_nh)
    return _tc_final(q[0, :n], q[1, :n], hs2, dinv, W2, b2r)


# speedup vs baseline: 1.0036x; 1.0036x over previous
import functools

import jax
import jax.numpy as jnp
from jax import lax
from jax.experimental import pallas as pl
from jax.experimental.pallas import tpu as pltpu
from jax.experimental.pallas import tpu_sc as plsc

CHUNK = 128
NUM_CORES = 2
NUM_SUBCORES = 16
NW = NUM_CORES * NUM_SUBCORES
N_PAD = 10240
N_PAD_DEG = 16384


def _sc_degree(dst2d, zeros_np):
    nch = dst2d.shape[0]
    cpw = nch // NW
    rps = N_PAD_DEG // NUM_SUBCORES
    mesh = plsc.VectorSubcoreMesh(core_axis_name="c", subcore_axis_name="s")

    @functools.partial(
        pl.kernel,
        mesh=mesh,
        out_type=jax.ShapeDtypeStruct((NUM_CORES * N_PAD_DEG,), jnp.float32),
        compiler_params=pltpu.CompilerParams(use_tc_tiling_on_sc=False),
        scratch_types=[
            pltpu.VMEM((cpw, CHUNK), jnp.int32),
            pltpu.VMEM((CHUNK,), jnp.float32),
            pltpu.VMEM_SHARED((N_PAD_DEG,), jnp.float32),
            pltpu.SemaphoreType.DMA,
        ],
    )
    def deg_kernel(dst_hbm, z_hbm, out_hbm, dst_v, ones_v, acc, sem):
        c = lax.axis_index("c")
        s = lax.axis_index("s")
        wid = s * NUM_CORES + c
        pltpu.sync_copy(z_hbm.at[pl.ds(s * rps, rps)], acc.at[pl.ds(s * rps, rps)])
        pltpu.sync_copy(dst_hbm.at[pl.ds(wid * cpw, cpw)], dst_v)
        for i in range(CHUNK // 16):
            ones_v[pl.ds(i * 16, 16)] = jnp.full((16,), 1.0, jnp.float32)
        plsc.subcore_barrier()

        def fire(j, carry):
            pltpu.make_async_copy(ones_v, acc.at[dst_v.at[j]], sem).start(add=True)
            return carry

        lax.fori_loop(0, cpw, fire, 0)

        def drain(j, carry):
            pltpu.make_async_copy(ones_v, acc.at[pl.ds(0, CHUNK)], sem).wait()
            return carry

        lax.fori_loop(0, cpw, drain, 0)
        plsc.subcore_barrier()
        pltpu.sync_copy(acc.at[pl.ds(s * rps, rps)],
                        out_hbm.at[pl.ds(c * N_PAD_DEG + s * rps, rps)])

    return deg_kernel(dst2d, zeros_np)


def _sc_aggregate(hs, src2d, dst2d, zeros_nf):
    n, f = hs.shape
    nch = src2d.shape[0]
    cpw = nch // NW
    epw = cpw * CHUNK
    rps = N_PAD // NUM_SUBCORES
    mesh = plsc.VectorSubcoreMesh(core_axis_name="c", subcore_axis_name="s")

    @functools.partial(
        pl.kernel,
        mesh=mesh,
        out_type=jax.ShapeDtypeStruct((NUM_CORES, N_PAD, f), jnp.float32),
        compiler_params=pltpu.CompilerParams(use_tc_tiling_on_sc=False),
        scratch_types=[
            pltpu.VMEM((cpw, CHUNK), jnp.int32),
            pltpu.VMEM((cpw, CHUNK), jnp.int32),
            pltpu.VMEM((cpw * CHUNK, f), jnp.float32),
            pltpu.VMEM_SHARED((N_PAD, f), jnp.float32),
            pltpu.SemaphoreType.DMA,
            pltpu.SemaphoreType.DMA,
        ],
    )
    def agg_kernel(hs_hbm, src_hbm, dst_hbm, z_hbm, out_hbm,
                   src_v, dst_v, rows_v, acc, gsem, ssem):
        c = lax.axis_index("c")
        s = lax.axis_index("s")
        wid = s * NUM_CORES + c
        pltpu.sync_copy(z_hbm.at[pl.ds(s * rps, rps)], acc.at[pl.ds(s * rps, rps)])
        pltpu.sync_copy(src_hbm.at[pl.ds(wid * cpw, cpw)], src_v)
        pltpu.sync_copy(dst_hbm.at[pl.ds(wid * cpw, cpw)], dst_v)
        plsc.subcore_barrier()

        def fire_gather(j, carry):
            pltpu.make_async_copy(
                hs_hbm.at[src_v.at[j]],
                rows_v.at[pl.ds(j * CHUNK, CHUNK), :], gsem).start()
            return carry

        lax.fori_loop(0, cpw, fire_gather, 0)

        def drain_gather(j, carry):
            pltpu.make_async_copy(hs_hbm.at[pl.ds(0, CHUNK), :],
                                  rows_v.at[pl.ds(0, CHUNK), :], gsem).wait()
            return carry

        lax.fori_loop(0, cpw, drain_gather, 0)

        def fire_scatter(j, carry):
            pltpu.make_async_copy(
                rows_v.at[pl.ds(j * CHUNK, CHUNK), :],
                acc.at[dst_v.at[j]], ssem).start(add=True)
            return carry

        lax.fori_loop(0, cpw, fire_scatter, 0)

        def drain_scatter(j, carry):
            pltpu.make_async_copy(rows_v.at[pl.ds(0, CHUNK), :],
                                  acc.at[pl.ds(0, CHUNK), :], ssem).wait()
            return carry

        lax.fori_loop(0, cpw, drain_scatter, 0)
        plsc.subcore_barrier()
        pltpu.sync_copy(acc.at[pl.ds(s * rps, rps), :],
                        out_hbm.at[c, pl.ds(s * rps, rps), :])

    return agg_kernel(hs, src2d, dst2d, zeros_nf)


def _tc_mm1(x, w1, block_n=2000):
    n, f_in = x.shape
    hid = w1.shape[1]
    grid = n // block_n

    def body(x_ref, w_ref, out_ref):
        out_ref[...] = jnp.dot(x_ref[...], w_ref[...],
                               preferred_element_type=jnp.float32)

    return pl.pallas_call(
        body,
        grid=(grid,),
        in_specs=[
            pl.BlockSpec((block_n, f_in), lambda i: (i, 0)),
            pl.BlockSpec((f_in, hid), lambda i: (0, 0)),
        ],
        out_specs=pl.BlockSpec((block_n, hid), lambda i: (i, 0)),
        out_shape=jax.ShapeDtypeStruct((n, hid), jnp.float32),
    )(x, w1)


def _tc_scale1(hw1, d0, d1, block_n=2000):
    n, hid = hw1.shape
    grid = n // block_n

    def body(hw_ref, d0_ref, d1_ref, hs_ref, dinv_ref):
        deg = d0_ref[...] + d1_ref[...] + 1.0
        dinv = lax.rsqrt(deg)
        hs_ref[...] = hw_ref[...] * dinv
        dinv_ref[...] = dinv

    return pl.pallas_call(
        body,
        grid=(grid,),
        in_specs=[
            pl.BlockSpec((block_n, hid), lambda i: (i, 0)),
            pl.BlockSpec((block_n, 1), lambda i: (i, 0)),
            pl.BlockSpec((block_n, 1), lambda i: (i, 0)),
        ],
        out_specs=[
            pl.BlockSpec((block_n, hid), lambda i: (i, 0)),
            pl.BlockSpec((block_n, 1), lambda i: (i, 0)),
        ],
        out_shape=[
            jax.ShapeDtypeStruct((n, hid), jnp.float32),
            jax.ShapeDtypeStruct((n, 1), jnp.float32),
        ],
    )(hw1, d0, d1)


def _tc_layer2(a0, a1, hs1, dinv, b1, block_n=2000):
    n, hid = hs1.shape
    grid = n // block_n

    def body(a0_ref, a1_ref, hs_ref, dinv_ref, b1_ref, out_ref):
        dinv = dinv_ref[...]
        t = dinv * (a0_ref[...] + a1_ref[...] + hs_ref[...]) + b1_ref[...]
        out_ref[...] = jnp.maximum(t, 0.0) * dinv

    return pl.pallas_call(
        body,
        grid=(grid,),
        in_specs=[
            pl.BlockSpec((block_n, hid), lambda i: (i, 0)),
            pl.BlockSpec((block_n, hid), lambda i: (i, 0)),
            pl.BlockSpec((block_n, hid), lambda i: (i, 0)),
            pl.BlockSpec((block_n, 1), lambda i: (i, 0)),
            pl.BlockSpec((1, hid), lambda i: (0, 0)),
        ],
        out_specs=pl.BlockSpec((block_n, hid), lambda i: (i, 0)),
        out_shape=jax.ShapeDtypeStruct((n, hid), jnp.float32),
    )(a0, a1, hs1, dinv, b1)


def _tc_final(q0, q1, hs2, dinv, w2, b2, block_n=2000):
    n, hid = hs2.shape
    c_out = w2.shape[1]
    grid = n // block_n

    def body(q0_ref, q1_ref, hs_ref, dinv_ref, w2_ref, b2_ref, out_ref):
        t = dinv_ref[...] * (q0_ref[...] + q1_ref[...] + hs_ref[...])
        logits = jnp.dot(t, w2_ref[...],
                         preferred_element_type=jnp.float32) + b2_ref[...]
        m = jnp.max(logits, axis=1, keepdims=True)
        lse = jnp.log(jnp.sum(jnp.exp(logits - m), axis=1, keepdims=True)) + m
        out_ref[...] = logits - lse

    return pl.pallas_call(
        body,
        grid=(grid,),
        in_specs=[
            pl.BlockSpec((block_n, hid), lambda i: (i, 0)),
            pl.BlockSpec((block_n, hid), lambda i: (i, 0)),
            pl.BlockSpec((block_n, hid), lambda i: (i, 0)),
            pl.BlockSpec((block_n, 1), lambda i: (i, 0)),
            pl.BlockSpec((hid, c_out), lambda i: (0, 0)),
            pl.BlockSpec((1, c_out), lambda i: (0, 0)),
        ],
        out_specs=pl.BlockSpec((block_n, c_out), lambda i: (i, 0)),
        out_shape=jax.ShapeDtypeStruct((n, c_out), jnp.float32),
    )(q0, q1, hs2, dinv, w2, b2)


def kernel(x, edge_index, W1, b1, W2, b2):
    n, f_in = x.shape
    e = edge_index.shape[1]
    hid = W1.shape[1]
    c_out = W2.shape[1]

    e_pad = ((e + CHUNK * NW - 1) // (CHUNK * NW)) * (CHUNK * NW)
    pad = e_pad - e
    src_full = jnp.concatenate([edge_index[0], jnp.zeros((pad,), jnp.int32)])
    dst_full = jnp.concatenate(
        [edge_index[1], jnp.full((pad,), n, jnp.int32)])
    src2d = src_full.reshape(e_pad // CHUNK, CHUNK)
    dst2d = dst_full.reshape(e_pad // CHUNK, CHUNK)

    zeros_np = jnp.zeros((N_PAD_DEG,), jnp.float32)
    zeros_nh = jnp.zeros((N_PAD, hid), jnp.float32)
    b1r = b1.reshape(1, hid)
    b2r = b2.reshape(1, c_out)

    hw1 = _tc_mm1(x, W1)
    degp = _sc_degree(dst2d, zeros_np)
    d0 = degp[:n].reshape(n, 1)
    d1 = degp[N_PAD_DEG:N_PAD_DEG + n].reshape(n, 1)

    hs1, dinv = _tc_scale1(hw1, d0, d1)
    p = _sc_aggregate(hs1, src2d, dst2d, zeros_nh)
    hs2 = _tc_layer2(p[0, :n], p[1, :n], hs1, dinv, b1r)
    q = _sc_aggregate(hs2, src2d, dst2d, zeros_nh)
    return _tc_final(q[0, :n], q[1, :n], hs2, dinv, W2, b2r)
